# Initial kernel scaffold; baseline (speedup 1.0000x reference)
import jax
import jax.numpy as jnp
from jax.experimental import pallas as pl


def kernel(conv0_w, conv0_b, conv0_gamma, conv0_beta, conv0_mean, conv0_var, conv1_w, conv1_b, conv1_gamma, conv1_beta, conv1_mean, conv1_var, conv11_w, conv11_b, conv11_gamma, conv11_beta, conv11_mean, conv11_var, conv2_w, conv2_b, conv2_gamma, conv2_beta, conv2_mean, conv2_var, conv21_w, conv21_b, conv21_gamma, conv21_beta, conv21_mean, conv21_var, conv3_w, conv3_b, conv3_gamma, conv3_beta, conv3_mean, conv3_var, conv31_w, conv31_b, conv31_gamma, conv31_beta, conv31_mean, conv31_var, conv4_w, conv4_b, conv4_gamma, conv4_beta, conv4_mean, conv4_var, conv41_w, conv41_b, conv41_gamma, conv41_beta, conv41_mean, conv41_var, cls_fc1_w, cls_fc1_b, cls_fc2_w, cls_fc2_b, x):
    raise NotImplementedError("write your pallas kernel here")


# speedup vs baseline: 3.4661x; 3.4661x over previous
import jax
import jax.numpy as jnp
from jax.experimental import pallas as pl
from jax.experimental.pallas import tpu as pltpu

_B = 32
_MXU_DT = jnp.bfloat16

_SPECS = {
    "c0":  (3, 2, (1, 1), (3, 1), (1, 1), True),
    "c1":  (3, 3, (1, 1), (0, 0), (1, 1), True),
    "c11": (3, 3, (1, 1), (2, 2), (1, 1), False),
    "c2":  (3, 3, (1, 1), (1, 1), (1, 1), True),
    "c21": (3, 3, (2, 2), (1, 1), (1, 1), True),
    "c3":  (3, 3, (1, 1), (1, 1), (1, 1), True),
    "c31": (3, 3, (2, 2), (1, 1), (2, 1), True),
    "c4":  (3, 3, (1, 1), (1, 1), (1, 1), True),
    "c41": (3, 3, (2, 2), (1, 1), (1, 1), True),
}


def _fold_bn(w, b, gamma, beta, mean, var, eps=1e-5):
    scale = gamma / jnp.sqrt(var + eps)
    w_f = w * scale[:, None, None, None]
    b_f = (b - mean) * scale + beta
    w2 = jnp.transpose(w_f, (2, 3, 1, 0))
    w2 = w2.reshape(-1, w.shape[0]).astype(_MXU_DT)
    return w2, b_f.reshape(1, -1)


def _hsel(t, ho, sh):
    if sh == 1:
        return t
    b, _, w, c = t.shape
    return t.reshape(b, ho, sh, w, c)[:, :, 0]


def _align(x, s, m, wlog):
    if s == 0 and x.shape[2] == (wlog - 1) * m + 1:
        return x
    return x[:, :, s:s + (wlog - 1) * m + 1, :]


def _conv(state, w2, bias, spec):
    x, m, s, wlog = state
    kh, kw, (sh, sw), (ph, pw), (dh, dw), relu = spec
    bsz, h, wb, c = x.shape
    ho = (h + 2 * ph - dh * (kh - 1) - 1) // sh + 1
    wo = (wlog + 2 * pw - dw * (kw - 1) - 1) // sw + 1
    m_out = m * sw
    wspan = (wo - 1) * m_out + 1
    b0 = s - pw * m
    pad_l = max(0, -b0)
    b0 += pad_l
    pad_r = max(0, b0 + (kw - 1) * dw * m + wspan - (wb + pad_l))
    need_h = (kh - 1) * dh + sh * ho
    x = jnp.pad(x, ((0, 0), (ph, need_h - h - ph), (pad_l, pad_r), (0, 0)))
    xb = x.astype(_MXU_DT)
    taps = [
        _hsel(xb[:, i * dh:i * dh + sh * ho, b0 + j * dw * m:b0 + j * dw * m + wspan, :],
              ho, sh)
        for i in range(kh) for j in range(kw)
    ]
    patches = jnp.concatenate(taps, axis=-1)
    acc = jax.lax.dot_general(patches, w2, (((3,), (0,)), ((), ())),
                              preferred_element_type=jnp.float32)
    acc = acc + bias.reshape(1, 1, 1, -1)
    if relu:
        acc = jnp.maximum(acc, 0.0)
    return acc, m_out, 0, wo


def _pool2(state):
    x, m, s, wlog = state
    b, h, wb, c = x.shape
    ho, wo = h // 2, wlog // 2
    xh = x[:, :ho * 2].reshape(b, ho, 2, wb, c)
    mx = jnp.maximum(xh[:, :, 0], xh[:, :, 1])
    mx = jnp.maximum(mx[:, :, :wb - m, :], mx[:, :, m:, :])
    return mx, 2 * m, s, wo


def _merge(state, state1):
    xp, mp, sp, wp = _pool2(state)
    x1, m1, s1, w1 = state1
    a = _align(xp, sp, mp, wp)
    b1 = _align(x1, s1, m1, w1)
    c = a.shape[-1]
    out = jnp.concatenate([b1[..., :c] + a, b1[..., c:]], axis=-1)
    return out, m1, 0, w1


def _net_kernel(x_ref,
                w0, b0, w1, b1, w11, b11, w2, b2, w21, b21,
                w3, b3, w31, b31, w4, b4, w41, b41,
                f1w, f1b, f2w, f2b, o_ref):
    x3 = x_ref[...]
    kh, kw, _, (ph, pw), _, _ = _SPECS["c0"]
    xp = jnp.pad(x3, ((0, 0), (ph, ph), (pw, pw))).astype(_MXU_DT)
    ho, wo = 38 - (kh - 1), 17 - (kw - 1)
    taps = [xp[:, i:i + ho, j:j + wo] for i in range(kh) for j in range(kw)]
    p0 = jnp.stack(taps, axis=-1)
    x0 = jax.lax.dot_general(p0, w0[...], (((3,), (0,)), ((), ())),
                             preferred_element_type=jnp.float32)
    x0 = jnp.maximum(x0 + b0[...].reshape(1, 1, 1, -1), 0.0)
    x = (x0, 1, 0, 16)

    x1 = _conv(x, w1[...], b1[...], _SPECS["c1"])
    x1 = _conv(x1, w11[...], b11[...], _SPECS["c11"])
    p, m, s, w = _pool2(x1)
    x = _merge(x, (jnp.maximum(p, 0.0), m, s, w))

    x1 = _conv(x, w2[...], b2[...], _SPECS["c2"])
    x1 = _conv(x1, w21[...], b21[...], _SPECS["c21"])
    x = _merge(x, x1)

    x1 = _conv(x, w3[...], b3[...], _SPECS["c3"])
    x1 = _conv(x1, w31[...], b31[...], _SPECS["c31"])
    x = _merge(x, x1)

    x1 = _conv(x, w4[...], b4[...], _SPECS["c4"])
    x1 = _conv(x1, w41[...], b41[...], _SPECS["c41"])
    xf, mf, sf, wf = _merge(x, x1)

    xa = _align(xf, sf, mf, wf)
    feat = jnp.maximum(xa[:, 0, 0, :], xa[:, 1, 0, :])
    h = jnp.dot(feat, f1w[...], preferred_element_type=jnp.float32) + f1b[...]
    h = jnp.maximum(h, 0.0)
    logits = jnp.dot(h, f2w[...], preferred_element_type=jnp.float32) + f2b[...]
    mx = jnp.max(logits, axis=-1, keepdims=True)
    e = jnp.exp(logits - mx)
    o_ref[...] = e / jnp.sum(e, axis=-1, keepdims=True)


def kernel(conv0_w, conv0_b, conv0_gamma, conv0_beta, conv0_mean, conv0_var,
           conv1_w, conv1_b, conv1_gamma, conv1_beta, conv1_mean, conv1_var,
           conv11_w, conv11_b, conv11_gamma, conv11_beta, conv11_mean, conv11_var,
           conv2_w, conv2_b, conv2_gamma, conv2_beta, conv2_mean, conv2_var,
           conv21_w, conv21_b, conv21_gamma, conv21_beta, conv21_mean, conv21_var,
           conv3_w, conv3_b, conv3_gamma, conv3_beta, conv3_mean, conv3_var,
           conv31_w, conv31_b, conv31_gamma, conv31_beta, conv31_mean, conv31_var,
           conv4_w, conv4_b, conv4_gamma, conv4_beta, conv4_mean, conv4_var,
           conv41_w, conv41_b, conv41_gamma, conv41_beta, conv41_mean, conv41_var,
           cls_fc1_w, cls_fc1_b, cls_fc2_w, cls_fc2_b, x):
    n = x.shape[0]
    folded = [
        _fold_bn(conv0_w, conv0_b, conv0_gamma, conv0_beta, conv0_mean, conv0_var),
        _fold_bn(conv1_w, conv1_b, conv1_gamma, conv1_beta, conv1_mean, conv1_var),
        _fold_bn(conv11_w, conv11_b, conv11_gamma, conv11_beta, conv11_mean, conv11_var),
        _fold_bn(conv2_w, conv2_b, conv2_gamma, conv2_beta, conv2_mean, conv2_var),
        _fold_bn(conv21_w, conv21_b, conv21_gamma, conv21_beta, conv21_mean, conv21_var),
        _fold_bn(conv3_w, conv3_b, conv3_gamma, conv3_beta, conv3_mean, conv3_var),
        _fold_bn(conv31_w, conv31_b, conv31_gamma, conv31_beta, conv31_mean, conv31_var),
        _fold_bn(conv4_w, conv4_b, conv4_gamma, conv4_beta, conv4_mean, conv4_var),
        _fold_bn(conv41_w, conv41_b, conv41_gamma, conv41_beta, conv41_mean, conv41_var),
    ]
    wargs = [a for wb in folded for a in wb]
    wargs += [cls_fc1_w.T, cls_fc1_b.reshape(1, -1),
              cls_fc2_w.T, cls_fc2_b.reshape(1, -1)]

    x3 = x.reshape(n, 32, 15)
    const = lambda *s: pl.BlockSpec(s, lambda b: tuple(0 for _ in s))
    wspecs = [const(*a.shape) for a in wargs]
    out = pl.pallas_call(
        _net_kernel,
        out_shape=jax.ShapeDtypeStruct((n, 5), jnp.float32),
        grid_spec=pltpu.PrefetchScalarGridSpec(
            num_scalar_prefetch=0,
            grid=(n // _B,),
            in_specs=[pl.BlockSpec((_B, 32, 15), lambda b: (b, 0, 0))] + wspecs,
            out_specs=pl.BlockSpec((_B, 5), lambda b: (b, 0)),
        ),
        compiler_params=pltpu.CompilerParams(dimension_semantics=("parallel",)),
    )(x3, *wargs)
    return out


# speedup vs baseline: 3.7266x; 1.0752x over previous
import jax
import jax.numpy as jnp
from jax.experimental import pallas as pl
from jax.experimental.pallas import tpu as pltpu

_B = 32
_MXU_DT = jnp.bfloat16

_SPECS = {
    "c0":  (3, 2, (1, 1), (3, 1), (1, 1), True),
    "c1":  (3, 3, (1, 1), (0, 0), (1, 1), True),
    "c11": (3, 3, (1, 1), (2, 2), (1, 1), False),
    "c2":  (3, 3, (1, 1), (1, 1), (1, 1), True),
    "c21": (3, 3, (2, 2), (1, 1), (1, 1), True),
    "c3":  (3, 3, (1, 1), (1, 1), (1, 1), True),
    "c31": (3, 3, (2, 2), (1, 1), (2, 1), True),
    "c4":  (3, 3, (1, 1), (1, 1), (1, 1), True),
    "c41": (3, 3, (2, 2), (1, 1), (1, 1), True),
}


def _fold_bn(w, b, gamma, beta, mean, var, eps=1e-5):
    scale = gamma / jnp.sqrt(var + eps)
    w_f = w * scale[:, None, None, None]
    b_f = (b - mean) * scale + beta
    w2 = jnp.transpose(w_f, (2, 3, 1, 0))
    w2 = w2.reshape(-1, w.shape[0]).astype(_MXU_DT)
    return w2, b_f.reshape(1, -1)


def _hsel(t, ho, sh):
    if sh == 1:
        return t
    b, _, w, c = t.shape
    return t.reshape(b, ho, sh, w, c)[:, :, 0]


def _align(x, s, m, wlog):
    if s == 0 and x.shape[2] == (wlog - 1) * m + 1:
        return x
    return x[:, :, s:s + (wlog - 1) * m + 1, :]


def _conv(state, w2, bias, spec):
    x, m, s, wlog = state
    kh, kw, (sh, sw), (ph, pw), (dh, dw), relu = spec
    bsz, h, wb, c = x.shape
    ho = (h + 2 * ph - dh * (kh - 1) - 1) // sh + 1
    wo = (wlog + 2 * pw - dw * (kw - 1) - 1) // sw + 1
    m_out = m * sw
    wspan = (wo - 1) * m_out + 1
    b0 = s - pw * m
    pad_l = max(0, -b0)
    b0 += pad_l
    pad_r = max(0, b0 + (kw - 1) * dw * m + wspan - (wb + pad_l))
    need_h = (kh - 1) * dh + sh * ho
    xb = jnp.pad(x, ((0, 0), (ph, need_h - h - ph), (pad_l, pad_r), (0, 0)))
    taps = [
        _hsel(xb[:, i * dh:i * dh + sh * ho, b0 + j * dw * m:b0 + j * dw * m + wspan, :],
              ho, sh)
        for i in range(kh) for j in range(kw)
    ]
    patches = jnp.concatenate(taps, axis=-1)
    acc = jax.lax.dot_general(patches, w2, (((3,), (0,)), ((), ())),
                              preferred_element_type=jnp.float32)
    acc = acc + bias.reshape(1, 1, 1, -1)
    if relu:
        acc = jnp.maximum(acc, 0.0)
    return acc.astype(_MXU_DT), m_out, 0, wo


def _pool2(state):
    x, m, s, wlog = state
    b, h, wb, c = x.shape
    ho, wo = h // 2, wlog // 2
    xh = x[:, :ho * 2].reshape(b, ho, 2, wb, c)
    mx = jnp.maximum(xh[:, :, 0], xh[:, :, 1])
    mx = jnp.maximum(mx[:, :, :wb - m, :], mx[:, :, m:, :])
    return mx, 2 * m, s, wo


def _merge(state, state1):
    xp, mp, sp, wp = _pool2(state)
    x1, m1, s1, w1 = state1
    a = _align(xp, sp, mp, wp)
    b1 = _align(x1, s1, m1, w1)
    c = a.shape[-1]
    out = jnp.concatenate([b1[..., :c] + a, b1[..., c:]], axis=-1)
    return out, m1, 0, w1


def _net_kernel(x_ref,
                w0, b0, w1, b1, w11, b11, w2, b2, w21, b21,
                w3, b3, w31, b31, w4, b4, w41, b41,
                f1w, f1b, f2w, f2b, o_ref):
    x3 = x_ref[...]
    kh, kw, _, (ph, pw), _, _ = _SPECS["c0"]
    xp = jnp.pad(x3, ((0, 0), (ph, ph), (pw, pw))).astype(_MXU_DT)
    ho, wo = 38 - (kh - 1), 17 - (kw - 1)
    taps = [xp[:, i:i + ho, j:j + wo] for i in range(kh) for j in range(kw)]
    p0 = jnp.stack(taps, axis=-1)
    x0 = jax.lax.dot_general(p0, w0[...], (((3,), (0,)), ((), ())),
                             preferred_element_type=jnp.float32)
    x0 = jnp.maximum(x0 + b0[...].reshape(1, 1, 1, -1), 0.0)
    x = (x0.astype(_MXU_DT), 1, 0, 16)

    x1 = _conv(x, w1[...], b1[...], _SPECS["c1"])
    x1 = _conv(x1, w11[...], b11[...], _SPECS["c11"])
    p, m, s, w = _pool2(x1)
    x = _merge(x, (jnp.maximum(p, jnp.asarray(0.0, p.dtype)), m, s, w))

    x1 = _conv(x, w2[...], b2[...], _SPECS["c2"])
    x1 = _conv(x1, w21[...], b21[...], _SPECS["c21"])
    x = _merge(x, x1)

    x1 = _conv(x, w3[...], b3[...], _SPECS["c3"])
    x1 = _conv(x1, w31[...], b31[...], _SPECS["c31"])
    x = _merge(x, x1)

    x1 = _conv(x, w4[...], b4[...], _SPECS["c4"])
    x1 = _conv(x1, w41[...], b41[...], _SPECS["c41"])
    xf, mf, sf, wf = _merge(x, x1)

    xa = _align(xf, sf, mf, wf)
    feat = jnp.maximum(xa[:, 0, 0, :], xa[:, 1, 0, :])
    feat = feat.astype(jnp.float32)
    h = jnp.dot(feat, f1w[...], preferred_element_type=jnp.float32) + f1b[...]
    h = jnp.maximum(h, 0.0)
    logits = jnp.dot(h, f2w[...], preferred_element_type=jnp.float32) + f2b[...]
    mx = jnp.max(logits, axis=-1, keepdims=True)
    e = jnp.exp(logits - mx)
    o_ref[...] = e / jnp.sum(e, axis=-1, keepdims=True)


def kernel(conv0_w, conv0_b, conv0_gamma, conv0_beta, conv0_mean, conv0_var,
           conv1_w, conv1_b, conv1_gamma, conv1_beta, conv1_mean, conv1_var,
           conv11_w, conv11_b, conv11_gamma, conv11_beta, conv11_mean, conv11_var,
           conv2_w, conv2_b, conv2_gamma, conv2_beta, conv2_mean, conv2_var,
           conv21_w, conv21_b, conv21_gamma, conv21_beta, conv21_mean, conv21_var,
           conv3_w, conv3_b, conv3_gamma, conv3_beta, conv3_mean, conv3_var,
           conv31_w, conv31_b, conv31_gamma, conv31_beta, conv31_mean, conv31_var,
           conv4_w, conv4_b, conv4_gamma, conv4_beta, conv4_mean, conv4_var,
           conv41_w, conv41_b, conv41_gamma, conv41_beta, conv41_mean, conv41_var,
           cls_fc1_w, cls_fc1_b, cls_fc2_w, cls_fc2_b, x):
    n = x.shape[0]
    folded = [
        _fold_bn(conv0_w, conv0_b, conv0_gamma, conv0_beta, conv0_mean, conv0_var),
        _fold_bn(conv1_w, conv1_b, conv1_gamma, conv1_beta, conv1_mean, conv1_var),
        _fold_bn(conv11_w, conv11_b, conv11_gamma, conv11_beta, conv11_mean, conv11_var),
        _fold_bn(conv2_w, conv2_b, conv2_gamma, conv2_beta, conv2_mean, conv2_var),
        _fold_bn(conv21_w, conv21_b, conv21_gamma, conv21_beta, conv21_mean, conv21_var),
        _fold_bn(conv3_w, conv3_b, conv3_gamma, conv3_beta, conv3_mean, conv3_var),
        _fold_bn(conv31_w, conv31_b, conv31_gamma, conv31_beta, conv31_mean, conv31_var),
        _fold_bn(conv4_w, conv4_b, conv4_gamma, conv4_beta, conv4_mean, conv4_var),
        _fold_bn(conv41_w, conv41_b, conv41_gamma, conv41_beta, conv41_mean, conv41_var),
    ]
    wargs = [a for wb in folded for a in wb]
    wargs += [cls_fc1_w.T, cls_fc1_b.reshape(1, -1),
              cls_fc2_w.T, cls_fc2_b.reshape(1, -1)]

    x3 = x.reshape(n, 32, 15)
    const = lambda *s: pl.BlockSpec(s, lambda b: tuple(0 for _ in s))
    wspecs = [const(*a.shape) for a in wargs]
    out = pl.pallas_call(
        _net_kernel,
        out_shape=jax.ShapeDtypeStruct((n, 5), jnp.float32),
        grid_spec=pltpu.PrefetchScalarGridSpec(
            num_scalar_prefetch=0,
            grid=(n // _B,),
            in_specs=[pl.BlockSpec((_B, 32, 15), lambda b: (b, 0, 0))] + wspecs,
            out_specs=pl.BlockSpec((_B, 5), lambda b: (b, 0)),
        ),
        compiler_params=pltpu.CompilerParams(dimension_semantics=("parallel",)),
    )(x3, *wargs)
    return out


# speedup vs baseline: 3.7277x; 1.0003x over previous
import jax
import jax.numpy as jnp
from jax.experimental import pallas as pl
from jax.experimental.pallas import tpu as pltpu

_B = 32
_MXU_DT = jnp.bfloat16

_SPECS = {
    "c0":  (3, 2, (1, 1), (3, 1), (1, 1), True),
    "c1":  (3, 3, (1, 1), (0, 0), (1, 1), True),
    "c11": (3, 3, (1, 1), (2, 2), (1, 1), False),
    "c2":  (3, 3, (1, 1), (1, 1), (1, 1), True),
    "c21": (3, 3, (2, 2), (1, 1), (1, 1), True),
    "c3":  (3, 3, (1, 1), (1, 1), (1, 1), True),
    "c31": (3, 3, (2, 2), (1, 1), (2, 1), True),
    "c4":  (3, 3, (1, 1), (1, 1), (1, 1), True),
    "c41": (3, 3, (2, 2), (1, 1), (1, 1), True),
}


def _fold_bn(w, b, gamma, beta, mean, var, eps=1e-5):
    scale = gamma / jnp.sqrt(var + eps)
    w_f = w * scale[:, None, None, None]
    b_f = (b - mean) * scale + beta
    w2 = jnp.transpose(w_f, (2, 3, 1, 0))
    w2 = w2.reshape(-1, w.shape[0]).astype(_MXU_DT)
    return w2, b_f.reshape(1, -1)


def _hsel(t, ho, sh):
    if sh == 1:
        return t
    b, _, w, c = t.shape
    return t.reshape(b, ho, sh, w, c)[:, :, 0]


def _align(x, s, m, wlog):
    if s == 0 and x.shape[2] == (wlog - 1) * m + 1:
        return x
    return x[:, :, s:s + (wlog - 1) * m + 1, :]


def _conv(state, w2, bias, spec, out_dt=_MXU_DT):
    x, m, s, wlog = state
    if x.dtype != _MXU_DT:
        x = x.astype(_MXU_DT)
    kh, kw, (sh, sw), (ph, pw), (dh, dw), relu = spec
    bsz, h, wb, c = x.shape
    ho = (h + 2 * ph - dh * (kh - 1) - 1) // sh + 1
    wo = (wlog + 2 * pw - dw * (kw - 1) - 1) // sw + 1
    m_out = m * sw
    wspan = (wo - 1) * m_out + 1
    b0 = s - pw * m
    pad_l = max(0, -b0)
    b0 += pad_l
    pad_r = max(0, b0 + (kw - 1) * dw * m + wspan - (wb + pad_l))
    need_h = (kh - 1) * dh + sh * ho
    xb = jnp.pad(x, ((0, 0), (ph, need_h - h - ph), (pad_l, pad_r), (0, 0)))
    taps = [
        _hsel(xb[:, i * dh:i * dh + sh * ho, b0 + j * dw * m:b0 + j * dw * m + wspan, :],
              ho, sh)
        for i in range(kh) for j in range(kw)
    ]
    patches = jnp.concatenate(taps, axis=-1)
    acc = jax.lax.dot_general(patches, w2, (((3,), (0,)), ((), ())),
                              preferred_element_type=jnp.float32)
    acc = acc + bias.reshape(1, 1, 1, -1)
    if relu:
        acc = jnp.maximum(acc, 0.0)
    return acc.astype(out_dt), m_out, 0, wo


def _pool2(state):
    x, m, s, wlog = state
    b, h, wb, c = x.shape
    ho, wo = h // 2, wlog // 2
    xh = x[:, :ho * 2].reshape(b, ho, 2, wb, c)
    mx = jnp.maximum(xh[:, :, 0], xh[:, :, 1])
    mx = jnp.maximum(mx[:, :, :wb - m, :], mx[:, :, m:, :])
    return mx, 2 * m, s, wo


def _merge(state, state1):
    xp, mp, sp, wp = _pool2(state)
    x1, m1, s1, w1 = state1
    a = _align(xp, sp, mp, wp)
    b1 = _align(x1, s1, m1, w1)
    c = a.shape[-1]
    out = jnp.concatenate([b1[..., :c] + a, b1[..., c:]], axis=-1)
    return out, m1, 0, w1


def _net_kernel(x_ref,
                w0, b0, w1, b1, w11, b11, w2, b2, w21, b21,
                w3, b3, w31, b31, w4, b4, w41, b41,
                f1w, f1b, f2w, f2b, o_ref):
    x3 = x_ref[...]
    kh, kw, _, (ph, pw), _, _ = _SPECS["c0"]
    xp = jnp.pad(x3, ((0, 0), (ph, ph), (pw, pw))).astype(_MXU_DT)
    ho, wo = 38 - (kh - 1), 17 - (kw - 1)
    taps = [xp[:, i:i + ho, j:j + wo] for i in range(kh) for j in range(kw)]
    p0 = jnp.stack(taps, axis=-1)
    x0 = jax.lax.dot_general(p0, w0[...], (((3,), (0,)), ((), ())),
                             preferred_element_type=jnp.float32)
    x0 = jnp.maximum(x0 + b0[...].reshape(1, 1, 1, -1), 0.0)
    x = (x0.astype(_MXU_DT), 1, 0, 16)

    x1 = _conv(x, w1[...], b1[...], _SPECS["c1"])
    x1 = _conv(x1, w11[...], b11[...], _SPECS["c11"])
    p, m, s, w = _pool2(x1)
    x = _merge(x, (jnp.maximum(p, jnp.asarray(0.0, p.dtype)), m, s, w))

    x1 = _conv(x, w2[...], b2[...], _SPECS["c2"])
    x1 = _conv(x1, w21[...], b21[...], _SPECS["c21"])
    x = _merge(x, x1)

    x1 = _conv(x, w3[...], b3[...], _SPECS["c3"])
    x1 = _conv(x1, w31[...], b31[...], _SPECS["c31"])
    x = _merge(x, x1)

    x1 = _conv(x, w4[...], b4[...], _SPECS["c4"], jnp.float32)
    x1 = _conv(x1, w41[...], b41[...], _SPECS["c41"], jnp.float32)
    xf, mf, sf, wf = _merge(x, x1)

    xa = _align(xf, sf, mf, wf)
    feat = jnp.maximum(xa[:, 0, 0, :], xa[:, 1, 0, :])
    feat = feat.astype(jnp.float32)
    h = jnp.dot(feat, f1w[...], preferred_element_type=jnp.float32) + f1b[...]
    h = jnp.maximum(h, 0.0)
    logits = jnp.dot(h, f2w[...], preferred_element_type=jnp.float32) + f2b[...]
    mx = jnp.max(logits, axis=-1, keepdims=True)
    e = jnp.exp(logits - mx)
    o_ref[...] = e / jnp.sum(e, axis=-1, keepdims=True)


def kernel(conv0_w, conv0_b, conv0_gamma, conv0_beta, conv0_mean, conv0_var,
           conv1_w, conv1_b, conv1_gamma, conv1_beta, conv1_mean, conv1_var,
           conv11_w, conv11_b, conv11_gamma, conv11_beta, conv11_mean, conv11_var,
           conv2_w, conv2_b, conv2_gamma, conv2_beta, conv2_mean, conv2_var,
           conv21_w, conv21_b, conv21_gamma, conv21_beta, conv21_mean, conv21_var,
           conv3_w, conv3_b, conv3_gamma, conv3_beta, conv3_mean, conv3_var,
           conv31_w, conv31_b, conv31_gamma, conv31_beta, conv31_mean, conv31_var,
           conv4_w, conv4_b, conv4_gamma, conv4_beta, conv4_mean, conv4_var,
           conv41_w, conv41_b, conv41_gamma, conv41_beta, conv41_mean, conv41_var,
           cls_fc1_w, cls_fc1_b, cls_fc2_w, cls_fc2_b, x):
    n = x.shape[0]
    folded = [
        _fold_bn(conv0_w, conv0_b, conv0_gamma, conv0_beta, conv0_mean, conv0_var),
        _fold_bn(conv1_w, conv1_b, conv1_gamma, conv1_beta, conv1_mean, conv1_var),
        _fold_bn(conv11_w, conv11_b, conv11_gamma, conv11_beta, conv11_mean, conv11_var),
        _fold_bn(conv2_w, conv2_b, conv2_gamma, conv2_beta, conv2_mean, conv2_var),
        _fold_bn(conv21_w, conv21_b, conv21_gamma, conv21_beta, conv21_mean, conv21_var),
        _fold_bn(conv3_w, conv3_b, conv3_gamma, conv3_beta, conv3_mean, conv3_var),
        _fold_bn(conv31_w, conv31_b, conv31_gamma, conv31_beta, conv31_mean, conv31_var),
        _fold_bn(conv4_w, conv4_b, conv4_gamma, conv4_beta, conv4_mean, conv4_var),
        _fold_bn(conv41_w, conv41_b, conv41_gamma, conv41_beta, conv41_mean, conv41_var),
    ]
    wargs = [a for wb in folded for a in wb]
    wargs += [cls_fc1_w.T, cls_fc1_b.reshape(1, -1),
              cls_fc2_w.T, cls_fc2_b.reshape(1, -1)]

    x3 = x.reshape(n, 32, 15)
    const = lambda *s: pl.BlockSpec(s, lambda b: tuple(0 for _ in s))
    wspecs = [const(*a.shape) for a in wargs]
    out = pl.pallas_call(
        _net_kernel,
        out_shape=jax.ShapeDtypeStruct((n, 5), jnp.float32),
        grid_spec=pltpu.PrefetchScalarGridSpec(
            num_scalar_prefetch=0,
            grid=(n // _B,),
            in_specs=[pl.BlockSpec((_B, 32, 15), lambda b: (b, 0, 0))] + wspecs,
            out_specs=pl.BlockSpec((_B, 5), lambda b: (b, 0)),
        ),
        compiler_params=pltpu.CompilerParams(dimension_semantics=("parallel",)),
    )(x3, *wargs)
    return out
